# baseline (speedup 1.0000x reference)
import functools

import jax
import jax.numpy as jnp
import numpy as np
from jax import lax
from jax.experimental import pallas as pl
from jax.experimental.pallas import tpu as pltpu
from jax.experimental.pallas import tpu_sc as plsc

_HEAD_ORDER = ['role_select', 'settler', 'builder', 'mayor', 'craftsman', 'trader', 'captain']
_HEAD_HIDDEN = [512, 256, 512, 512, 128, 256, 512]
_PHASE_TO_HEADIDX = np.array([1, 3, 2, 4, 5, 6, 6, 0, 0], dtype=np.int32)

_BLK_T = 512
_BLK_H = 256
_NB_H = 72
_NP = _NB_H * _BLK_H
_AMP = 256


def _bdot(a, b):
    return jax.lax.dot(a.astype(jnp.bfloat16), b, preferred_element_type=jnp.float32)


def _bdot16(a, b):
    return jax.lax.dot(a, b, preferred_element_type=jnp.float32)


def _normalize(x, eps=1e-5):
    m = jnp.mean(x, axis=-1, keepdims=True)
    v = jnp.mean(x * x, axis=-1, keepdims=True) - m * m
    return (x - m) * jax.lax.rsqrt(v + eps)


def _trunk_body(
    x_ref, ph_ref,
    pe_tab_ref, wx_ref, wp_ref, be_ref, ge_ref, bee_ref,
    bw1_ref, bb1_ref, bw2_ref, bb2_ref,
    cw1_ref, cb1_ref, cw2_ref, cb2_ref,
    nrm_ref, val_ref,
):
    f32 = jnp.float32
    blk = x_ref.shape[0]

    ph = ph_ref[...]
    iota9 = jax.lax.broadcasted_iota(jnp.int32, (blk, 9), 1)
    oh9 = (ph == iota9).astype(f32)
    pe = jnp.dot(oh9, pe_tab_ref[...])

    u = _bdot(x_ref[...], wx_ref[...]) + _bdot(pe, wp_ref[...]) + be_ref[...]
    h = jax.nn.relu(_normalize(u) * ge_ref[...] + bee_ref[...])

    for i in range(3):
        t = _normalize(h)
        t = jax.nn.relu(_bdot(t, bw1_ref[i]) + bb1_ref[i])
        t = jax.nn.relu(_bdot(t, bw2_ref[i]) + bb2_ref[i])
        h = h + t

    nrm = _normalize(h).astype(jnp.bfloat16)
    nrm_ref[...] = nrm

    v = jax.nn.relu(_bdot16(nrm, cw1_ref[...]) + cb1_ref[...])
    val_ref[...] = _bdot(v, cw2_ref[...]) + cb2_ref[...]


def _trunk(x, ph2, params):
    B, OBS = x.shape
    H = params['embed']['W'].shape[1]
    PE = params['phase_embed'].shape[1]
    nb = B // _BLK_T
    bf16 = jnp.bfloat16

    e = params['embed']
    wx = e['W'][:OBS].astype(bf16)
    wp = e['W'][OBS:].astype(bf16)
    bw1 = jnp.stack([b['g'][:, None] * b['W1'] for b in params['blocks']]).astype(bf16)
    bb1 = jnp.stack([b['b1'] + b['be'] @ b['W1'] for b in params['blocks']])
    bw2 = jnp.stack([b['W2'] for b in params['blocks']]).astype(bf16)
    bb2 = jnp.stack([b['b2'] for b in params['blocks']])
    c = params['critic']
    cw1 = (c['g'][:, None] * c['W1']).astype(bf16)
    cb1 = c['b1'] + c['be'] @ c['W1']

    row = lambda w, d=None: pl.BlockSpec((_BLK_T, w), lambda i: (i, 0))
    full = lambda *shape: pl.BlockSpec(shape, lambda i: (0,) * len(shape))

    nrm, val = pl.pallas_call(
        _trunk_body,
        grid=(nb,),
        in_specs=[
            row(OBS), row(1),
            full(9, PE), full(OBS, H), full(PE, H), full(H), full(H), full(H),
            full(3, H, H), full(3, H), full(3, H, H), full(3, H),
            full(H, H), full(H), full(H, 1), full(1),
        ],
        out_specs=[row(H), row(1)],
        out_shape=[
            jax.ShapeDtypeStruct((B, H), bf16),
            jax.ShapeDtypeStruct((B, 1), jnp.float32),
        ],
    )(
        x, ph2,
        params['phase_embed'], wx, wp, e['b'], e['g'], e['be'],
        bw1, bb1, bw2, bb2,
        cw1, cb1, c['W2'].astype(bf16), c['b2'],
    )
    return nrm, val


def _routing(phase_ids, action, B):
    i32 = jnp.int32
    hid = jnp.asarray(_PHASE_TO_HEADIDX)[phase_ids]
    oh = (hid[:, None] == jnp.arange(7)[None, :]).astype(i32)
    counts = jnp.sum(oh, axis=0)
    starts = jnp.cumsum(counts) - counts
    cap = ((counts + _BLK_H - 1) // _BLK_H) * _BLK_H
    pstart = jnp.cumsum(cap) - cap

    perm = jnp.argsort(hid)
    pos = jnp.arange(_NP, dtype=i32)
    h_slot = jnp.sum((pos[None, :] >= pstart[:, None]).astype(i32), axis=0) - 1
    h_slot = jnp.clip(h_slot, 0, 6)
    ps = pstart[h_slot]
    j = pos - ps
    cnt = counts[h_slot]
    valid = j < cnt
    src_pos = starts[h_slot] + jnp.clip(j, 0, jnp.maximum(cnt - 1, 0))
    src_pos = jnp.clip(src_pos, 0, B - 1)
    ridx = perm[src_pos].astype(i32)
    bh = h_slot[:: _BLK_H].astype(i32)
    ridx_out = jnp.where(valid, ridx, B)
    act_s = action.astype(i32)[ridx].reshape(_NP, 1)
    return ridx, ridx_out, bh, act_s


def _sc_gather(nrm3, am3, ridx):
    info = plsc.get_sparse_core_info()
    nw = info.num_cores * info.num_subcores
    np_w = _NP // nw
    ch = 192
    nch = np_w // ch
    bf16 = jnp.bfloat16
    mesh = plsc.VectorSubcoreMesh(core_axis_name="c", subcore_axis_name="s")

    @functools.partial(
        pl.kernel, mesh=mesh,
        out_type=[
            jax.ShapeDtypeStruct((_NP, 2, 128), jnp.int32),
            jax.ShapeDtypeStruct((_NP, 1, 128), jnp.int32),
        ],
        scratch_types=[
            pltpu.VMEM((np_w,), jnp.int32),
            pltpu.VMEM((ch, 2, 128), jnp.int32),
            pltpu.VMEM((ch, 1, 128), jnp.int32),
            pltpu.SemaphoreType.DMA,
            pltpu.SemaphoreType.DMA,
        ],
    )
    def gk(nrm_hbm, am_hbm, idx_hbm, nrm_out, am_out, idx_v, nbuf, abuf, s1, s2):
        wid = lax.axis_index("s") * info.num_cores + lax.axis_index("c")
        base = wid * np_w
        pltpu.sync_copy(idx_hbm.at[pl.ds(base, np_w)], idx_v)
        for ci in range(nch):
            off = ci * ch
            c1 = pltpu.async_copy(nrm_hbm.at[idx_v.at[pl.ds(off, ch)]], nbuf, s1)
            c2 = pltpu.async_copy(am_hbm.at[idx_v.at[pl.ds(off, ch)]], abuf, s2)
            c1.wait()
            c2.wait()
            pltpu.sync_copy(nbuf, nrm_out.at[pl.ds(base + off, ch)])
            pltpu.sync_copy(abuf, am_out.at[pl.ds(base + off, ch)])

    return gk(nrm3, am3, ridx)


def _heads_body(bh_ref, nrm_ref, am_ref, act_ref,
                w1_ref, b1_ref, w2_ref, b2_ref,
                logp_ref, ent_ref):
    f32 = jnp.float32
    blk = nrm_ref.shape[0]
    ACT = w2_ref.shape[2]

    t = nrm_ref[...]
    h1 = jnp.maximum(_bdot16(t, w1_ref[0]) + b1_ref[0], 0.0)
    logits = _bdot16(h1.astype(jnp.bfloat16), w2_ref[0]) + b2_ref[0]

    am = am_ref[...][:, :ACT]
    masked = jnp.where(am > 0.5, logits, f32(-1e8))
    mx = jnp.max(masked, axis=-1, keepdims=True)
    z = masked - mx
    ez = jnp.exp(z)
    s = jnp.sum(ez, axis=-1, keepdims=True)
    logp = z - jnp.log(s)

    act = act_ref[...]
    iota_a = jax.lax.broadcasted_iota(jnp.int32, (blk, ACT), 1)
    oh_a = (act == iota_a).astype(f32)
    logp_ref[...] = jnp.sum(logp * oh_a, axis=-1, keepdims=True)
    probs = ez / s
    ent_ref[...] = -jnp.sum(probs * logp, axis=-1, keepdims=True)


def _heads(nrm_s, am_s, act_s, bh, params):
    H = 512
    ACT = 200
    bf16 = jnp.bfloat16
    hp = params['heads']

    def padw1(n, hh):
        w = hp[n]['g'][:, None] * hp[n]['W1']
        return jnp.pad(w, ((0, 0), (0, H - hh)))

    def padb1(n, hh):
        b = hp[n]['b1'] + hp[n]['be'] @ hp[n]['W1']
        return jnp.pad(b, (0, H - hh))

    def padw2(n, hh):
        return jnp.pad(hp[n]['W2'], ((0, H - hh), (0, 0)))

    w1s = jnp.stack([padw1(n, hh) for n, hh in zip(_HEAD_ORDER, _HEAD_HIDDEN)]).astype(bf16)
    b1s = jnp.stack([padb1(n, hh) for n, hh in zip(_HEAD_ORDER, _HEAD_HIDDEN)])[:, None, :]
    w2s = jnp.stack([padw2(n, hh) for n, hh in zip(_HEAD_ORDER, _HEAD_HIDDEN)]).astype(bf16)
    b2s = jnp.stack([hp[n]['b2'] for n in _HEAD_ORDER])[:, None, :]

    grid_spec = pltpu.PrefetchScalarGridSpec(
        num_scalar_prefetch=1,
        grid=(_NB_H,),
        in_specs=[
            pl.BlockSpec((_BLK_H, H), lambda i, bh_r: (i, 0)),
            pl.BlockSpec((_BLK_H, _AMP), lambda i, bh_r: (i, 0)),
            pl.BlockSpec((_BLK_H, 1), lambda i, bh_r: (i, 0)),
            pl.BlockSpec((1, H, H), lambda i, bh_r: (bh_r[i], 0, 0)),
            pl.BlockSpec((1, 1, H), lambda i, bh_r: (bh_r[i], 0, 0)),
            pl.BlockSpec((1, H, ACT), lambda i, bh_r: (bh_r[i], 0, 0)),
            pl.BlockSpec((1, 1, ACT), lambda i, bh_r: (bh_r[i], 0, 0)),
        ],
        out_specs=[
            pl.BlockSpec((_BLK_H, 1), lambda i, bh_r: (i, 0)),
            pl.BlockSpec((_BLK_H, 1), lambda i, bh_r: (i, 0)),
        ],
    )
    logp_s, ent_s = pl.pallas_call(
        _heads_body,
        grid_spec=grid_spec,
        out_shape=[
            jax.ShapeDtypeStruct((_NP, 1), jnp.float32),
            jax.ShapeDtypeStruct((_NP, 1), jnp.float32),
        ],
    )(bh, nrm_s, am_s, act_s, w1s, b1s, w2s, b2s)
    return logp_s, ent_s


@jax.jit
def _run(x, action_mask, phase_ids, action, params):
    B = x.shape[0]
    ACT = action_mask.shape[1]
    ph2 = phase_ids.astype(jnp.int32).reshape(B, 1)

    nrm, val = _trunk(x, ph2, params)

    ridx, ridx_out, bh, act_s = _routing(phase_ids, action, B)

    bf16 = jnp.bfloat16
    am_b = jnp.pad(action_mask, ((0, 0), (0, _AMP - ACT))).astype(bf16)
    am3 = jax.lax.bitcast_convert_type(am_b.reshape(B, 128, 2), jnp.int32).reshape(B, 1, 128)
    nrm3 = jax.lax.bitcast_convert_type(nrm.reshape(B, 256, 2), jnp.int32).reshape(B, 2, 128)
    nrm_s3, am_s3 = _sc_gather(nrm3, am3, ridx)

    nrm_s = jax.lax.bitcast_convert_type(
        nrm_s3.reshape(_NP, 256), bf16).reshape(_NP, 512)
    am_s = jax.lax.bitcast_convert_type(
        am_s3.reshape(_NP, 128), bf16).reshape(_NP, _AMP)
    logp_s, ent_s = _heads(nrm_s, am_s, act_s, bh, params)

    f32 = jnp.float32
    logp = jnp.zeros((B,), f32).at[ridx_out].set(logp_s[:, 0], mode='drop')
    ent = jnp.zeros((B,), f32).at[ridx_out].set(ent_s[:, 0], mode='drop')
    return action, logp, ent, val


def kernel(x, action_mask, phase_ids, action, params):
    return _run(x, action_mask, phase_ids, action, params)


# speedup vs baseline: 2.8714x; 2.8714x over previous
import functools

import jax
import jax.numpy as jnp
import numpy as np
from jax.experimental import pallas as pl
from jax.experimental.pallas import tpu as pltpu

_HEAD_ORDER = ['role_select', 'settler', 'builder', 'mayor', 'craftsman', 'trader', 'captain']
_HEAD_HIDDEN = [512, 256, 512, 512, 128, 256, 512]
_PHASE_TO_HEADIDX = np.array([1, 3, 2, 4, 5, 6, 6, 0, 0], dtype=np.int32)
_OFFS = np.concatenate([[0], np.cumsum(_HEAD_HIDDEN)])
_HSUM = int(_OFFS[-1])


def _bdot(a, b):
    return jax.lax.dot(a.astype(jnp.bfloat16), b,
                       preferred_element_type=jnp.float32)


def _bdot16(a, b):
    return jax.lax.dot(a, b, preferred_element_type=jnp.float32)


def _normalize(x, eps=1e-5):
    m = jnp.mean(x, axis=-1, keepdims=True)
    v = jnp.mean(x * x, axis=-1, keepdims=True) - m * m
    return (x - m) * jax.lax.rsqrt(v + eps)


def _fused_body(
    x_ref, ph_ref, act_ref, amask_ref,
    pe_tab_ref, wx_ref, wp_ref, be_ref, ge_ref, bee_ref,
    bw1_ref, bb1_ref, bw2_ref, bb2_ref,
    cw1_ref, cb1_ref, cw2_ref, cb2_ref,
    hw1_ref, hb1_ref, hw2_ref, hb2_ref, p2h_ref,
    logp_ref, ent_ref, val_ref,
):
    f32 = jnp.float32
    blk = x_ref.shape[0]

    ph = ph_ref[...]
    iota9 = jax.lax.broadcasted_iota(jnp.int32, (blk, 9), 1)
    oh9 = (ph == iota9).astype(f32)
    pe = jnp.dot(oh9, pe_tab_ref[...])

    u = _bdot(x_ref[...], wx_ref[...]) + _bdot(pe, wp_ref[...]) + be_ref[...]
    h = jax.nn.relu(_normalize(u) * ge_ref[...] + bee_ref[...])

    for i in range(3):
        t = _normalize(h)
        t = jax.nn.relu(_bdot(t, bw1_ref[i]) + bb1_ref[i])
        t = jax.nn.relu(_bdot(t, bw2_ref[i]) + bb2_ref[i])
        h = h + t

    nrm = _normalize(h).astype(jnp.bfloat16)

    v = jax.nn.relu(_bdot16(nrm, cw1_ref[...]) + cb1_ref[...])
    val_ref[...] = _bdot(v, cw2_ref[...]) + cb2_ref[...]

    h1 = _bdot16(nrm, hw1_ref[...]) + hb1_ref[...]

    hid = jnp.dot(oh9, p2h_ref[...]).astype(jnp.int32)
    cols = jax.lax.broadcasted_iota(jnp.int32, (1, _HSUM), 1)
    seg = jnp.zeros((1, _HSUM), jnp.int32)
    for off in _OFFS[1:-1]:
        seg = seg + (cols >= int(off)).astype(jnp.int32)
    h1m = jnp.where(seg == hid, jax.nn.relu(h1).astype(jnp.bfloat16),
                    jnp.bfloat16(0))

    logits = _bdot16(h1m, hw2_ref[...])
    iota7 = jax.lax.broadcasted_iota(jnp.int32, (blk, 7), 1)
    oh7 = (iota7 == hid).astype(f32)
    logits = logits + jnp.dot(oh7, hb2_ref[...])

    masked = jnp.where(amask_ref[...] > 0.5, logits, f32(-1e8))
    mx = jnp.max(masked, axis=-1, keepdims=True)
    z = masked - mx
    ez = jnp.exp(z)
    s = jnp.sum(ez, axis=-1, keepdims=True)
    logp = z - jnp.log(s)

    act = act_ref[...]
    iota_a = jax.lax.broadcasted_iota(jnp.int32, (blk, logits.shape[1]), 1)
    oh_a = (act == iota_a).astype(f32)
    logp_ref[...] = jnp.sum(logp * oh_a, axis=-1, keepdims=True)
    probs = ez / s
    ent_ref[...] = -jnp.sum(probs * logp, axis=-1, keepdims=True)


@functools.partial(jax.jit, static_argnames=())
def _run(x, action_mask, phase_ids, action, params):
    B, OBS = x.shape
    ACT = action_mask.shape[1]
    H = params['embed']['W'].shape[1]
    PE = params['phase_embed'].shape[1]
    BLK = 512
    nb = B // BLK

    bf16 = jnp.bfloat16
    e = params['embed']
    wx = e['W'][:OBS].astype(bf16)
    wp = e['W'][OBS:].astype(bf16)

    bw1 = jnp.stack([blk['g'][:, None] * blk['W1'] for blk in params['blocks']]).astype(bf16)
    bb1 = jnp.stack([blk['b1'] + blk['be'] @ blk['W1'] for blk in params['blocks']])
    bw2 = jnp.stack([blk['W2'] for blk in params['blocks']]).astype(bf16)
    bb2 = jnp.stack([blk['b2'] for blk in params['blocks']])

    c = params['critic']
    cw1 = (c['g'][:, None] * c['W1']).astype(bf16)
    cb1 = c['b1'] + c['be'] @ c['W1']

    hp = params['heads']
    hw1 = jnp.concatenate(
        [hp[n]['g'][:, None] * hp[n]['W1'] for n in _HEAD_ORDER], axis=1).astype(bf16)
    hb1 = jnp.concatenate(
        [hp[n]['b1'] + hp[n]['be'] @ hp[n]['W1'] for n in _HEAD_ORDER])
    hw2 = jnp.concatenate([hp[n]['W2'] for n in _HEAD_ORDER], axis=0).astype(bf16)
    hb2 = jnp.stack([hp[n]['b2'] for n in _HEAD_ORDER])

    ph2 = phase_ids.astype(jnp.int32).reshape(B, 1)
    act2 = action.astype(jnp.int32).reshape(B, 1)

    row_spec = lambda w: pl.BlockSpec((BLK, w), lambda i: (i, 0))
    full = lambda *shape: pl.BlockSpec(shape, lambda i: (0,) * len(shape))

    out_shapes = [
        jax.ShapeDtypeStruct((B, 1), jnp.float32),
        jax.ShapeDtypeStruct((B, 1), jnp.float32),
        jax.ShapeDtypeStruct((B, 1), jnp.float32),
    ]
    logp, ent, val = pl.pallas_call(
        _fused_body,
        grid=(nb,),
        in_specs=[
            row_spec(OBS), row_spec(1), row_spec(1), row_spec(ACT),
            full(9, PE), full(OBS, H), full(PE, H), full(H), full(H), full(H),
            full(3, H, H), full(3, H), full(3, H, H), full(3, H),
            full(H, H), full(H), full(H, 1), full(1),
            full(H, _HSUM), full(_HSUM), full(_HSUM, ACT), full(7, ACT),
            full(9, 1),
        ],
        out_specs=[row_spec(1), row_spec(1), row_spec(1)],
        out_shape=out_shapes,
    )(
        x, ph2, act2, action_mask,
        params['phase_embed'], wx, wp, e['b'], e['g'], e['be'],
        bw1, bb1, bw2, bb2,
        cw1, cb1, c['W2'].astype(bf16), c['b2'],
        hw1, hb1, hw2, hb2,
        jnp.asarray(_PHASE_TO_HEADIDX.astype(np.float32)[:, None]),
    )
    return action, logp[:, 0], ent[:, 0], val


def kernel(x, action_mask, phase_ids, action, params):
    return _run(x, action_mask, phase_ids, action, params)


# speedup vs baseline: 3.0988x; 1.0792x over previous
import functools

import jax
import jax.numpy as jnp
import numpy as np
from jax.experimental import pallas as pl
from jax.experimental.pallas import tpu as pltpu

_HEAD_ORDER = ['role_select', 'settler', 'builder', 'mayor', 'craftsman', 'trader', 'captain']
_HEAD_HIDDEN = [512, 256, 512, 512, 128, 256, 512]
_PHASE_TO_HEADIDX = np.array([1, 3, 2, 4, 5, 6, 6, 0, 0], dtype=np.int32)
_OFFS = np.concatenate([[0], np.cumsum(_HEAD_HIDDEN)])
_HSUM = int(_OFFS[-1])
_H = 512
_ACT = 200
_OBS = 210


def _bdot16(a, b):
    return jax.lax.dot(a, b, preferred_element_type=jnp.float32)


def _bdot(a, b):
    return jax.lax.dot(a.astype(jnp.bfloat16), b, preferred_element_type=jnp.float32)


def _normalize(x, eps=1e-5):
    m = jnp.mean(x, axis=-1, keepdims=True)
    v = jnp.mean(x * x, axis=-1, keepdims=True) - m * m
    return (x - m) * jax.lax.rsqrt(v + eps)


def _prep_body(*refs):
    bf16 = jnp.bfloat16
    (ew_ref,
     b1g_ref, b1b_ref, b1w1_ref, b1b1_ref, b1w2_ref,
     b2g_ref, b2b_ref, b2w1_ref, b2b1_ref, b2w2_ref,
     b3g_ref, b3b_ref, b3w1_ref, b3b1_ref, b3w2_ref,
     cg_ref, cb_ref, cw1_ref, cb1_ref, cw2_ref,
     h0g_ref, h0b_ref, h0w1_ref, h0b1_ref, h0w2_ref, h0b2_ref,
     h1g_ref, h1b_ref, h1w1_ref, h1b1_ref, h1w2_ref, h1b2_ref,
     h2g_ref, h2b_ref, h2w1_ref, h2b1_ref, h2w2_ref, h2b2_ref,
     h3g_ref, h3b_ref, h3w1_ref, h3b1_ref, h3w2_ref, h3b2_ref,
     h4g_ref, h4b_ref, h4w1_ref, h4b1_ref, h4w2_ref, h4b2_ref,
     h5g_ref, h5b_ref, h5w1_ref, h5b1_ref, h5w2_ref, h5b2_ref,
     h6g_ref, h6b_ref, h6w1_ref, h6b1_ref, h6w2_ref, h6b2_ref,
     wx_o, wp_o,
     bw1_o1, bb1_o1, bw2_o1,
     bw1_o2, bb1_o2, bw2_o2,
     bw1_o3, bb1_o3, bw2_o3,
     cw1_o, cb1_o, cw2_o,
     hw1_o, hb1_o, hw2_o, hb2_o) = refs

    ew = ew_ref[...]
    wx_o[...] = ew[:_OBS].astype(bf16)
    wp_o[...] = ew[_OBS:].astype(bf16)

    for (g_r, b_r, w1_r, b1_r, w2_r, w1_o, b1_o, w2_o) in (
        (b1g_ref, b1b_ref, b1w1_ref, b1b1_ref, b1w2_ref, bw1_o1, bb1_o1, bw2_o1),
        (b2g_ref, b2b_ref, b2w1_ref, b2b1_ref, b2w2_ref, bw1_o2, bb1_o2, bw2_o2),
        (b3g_ref, b3b_ref, b3w1_ref, b3b1_ref, b3w2_ref, bw1_o3, bb1_o3, bw2_o3),
    ):
        w1 = w1_r[...]
        w1_o[...] = (g_r[...][:, None] * w1).astype(bf16)
        b1_o[...] = b1_r[...][None, :] + jnp.dot(b_r[...][None, :], w1)
        w2_o[...] = w2_r[...].astype(bf16)

    cw1 = cw1_ref[...]
    cw1_o[...] = (cg_ref[...][:, None] * cw1).astype(bf16)
    cb1_o[...] = cb1_ref[...][None, :] + jnp.dot(cb_ref[...][None, :], cw1)
    cw2_o[...] = cw2_ref[...].astype(bf16)

    heads = (
        (h0g_ref, h0b_ref, h0w1_ref, h0b1_ref, h0w2_ref, h0b2_ref),
        (h1g_ref, h1b_ref, h1w1_ref, h1b1_ref, h1w2_ref, h1b2_ref),
        (h2g_ref, h2b_ref, h2w1_ref, h2b1_ref, h2w2_ref, h2b2_ref),
        (h3g_ref, h3b_ref, h3w1_ref, h3b1_ref, h3w2_ref, h3b2_ref),
        (h4g_ref, h4b_ref, h4w1_ref, h4b1_ref, h4w2_ref, h4b2_ref),
        (h5g_ref, h5b_ref, h5w1_ref, h5b1_ref, h5w2_ref, h5b2_ref),
        (h6g_ref, h6b_ref, h6w1_ref, h6b1_ref, h6w2_ref, h6b2_ref),
    )
    for k, (g_r, b_r, w1_r, b1_r, w2_r, b2_r) in enumerate(heads):
        off, hh = int(_OFFS[k]), _HEAD_HIDDEN[k]
        w1 = w1_r[...]
        hw1_o[:, off:off + hh] = (g_r[...][:, None] * w1).astype(bf16)
        hb1_o[:, off:off + hh] = b1_r[...][None, :] + jnp.dot(b_r[...][None, :], w1)
        hw2_o[off:off + hh, :] = w2_r[...].astype(bf16)
        hb2_o[k:k + 1, :] = b2_r[...][None, :]
    hb2_o[7:8, :] = jnp.zeros((1, _ACT), jnp.float32)


def _prep(params):
    bf16 = jnp.bfloat16
    f32 = jnp.float32
    e = params['embed']
    ins = [e['W']]
    for b in params['blocks']:
        ins += [b['g'], b['be'], b['W1'], b['b1'], b['W2']]
    c = params['critic']
    ins += [c['g'], c['be'], c['W1'], c['b1'], c['W2']]
    for n in _HEAD_ORDER:
        hp = params['heads'][n]
        ins += [hp['g'], hp['be'], hp['W1'], hp['b1'], hp['W2'], hp['b2']]

    out_shape = [
        jax.ShapeDtypeStruct((_OBS, _H), bf16),
        jax.ShapeDtypeStruct((16, _H), bf16),
    ]
    for _ in range(3):
        out_shape += [
            jax.ShapeDtypeStruct((_H, _H), bf16),
            jax.ShapeDtypeStruct((1, _H), f32),
            jax.ShapeDtypeStruct((_H, _H), bf16),
        ]
    out_shape += [
        jax.ShapeDtypeStruct((_H, _H), bf16),
        jax.ShapeDtypeStruct((1, _H), f32),
        jax.ShapeDtypeStruct((_H, 1), bf16),
        jax.ShapeDtypeStruct((_H, _HSUM), bf16),
        jax.ShapeDtypeStruct((1, _HSUM), f32),
        jax.ShapeDtypeStruct((_HSUM, _ACT), bf16),
        jax.ShapeDtypeStruct((8, _ACT), f32),
    ]
    return pl.pallas_call(
        _prep_body,
        out_shape=out_shape,
    )(*ins)


def _fused_body(
    x_ref, ph_ref, act_ref, amask_ref,
    pe_tab_ref, wx_ref, wp_ref, be_ref, ge_ref, bee_ref,
    bw1_1, bb1_1, bw2_1, bb2_1,
    bw1_2, bb1_2, bw2_2, bb2_2,
    bw1_3, bb1_3, bw2_3, bb2_3,
    cw1_ref, cb1_ref, cw2_ref, cb2_ref,
    hw1_ref, hb1_ref, hw2_ref, hb2_ref, p2h_ref,
    logp_ref, ent_ref, val_ref,
):
    f32 = jnp.float32
    bf16 = jnp.bfloat16
    blk = x_ref.shape[0]

    ph = ph_ref[...]
    iota9 = jax.lax.broadcasted_iota(jnp.int32, (blk, 9), 1)
    oh9 = (ph == iota9).astype(f32)
    pe = jnp.dot(oh9, pe_tab_ref[...])

    u = _bdot(x_ref[...], wx_ref[...]) + _bdot(pe, wp_ref[...]) + be_ref[...]
    h = jax.nn.relu(_normalize(u) * ge_ref[...] + bee_ref[...])

    for (w1, b1, w2, b2) in (
        (bw1_1, bb1_1, bw2_1, bb2_1),
        (bw1_2, bb1_2, bw2_2, bb2_2),
        (bw1_3, bb1_3, bw2_3, bb2_3),
    ):
        t = _normalize(h)
        t = jax.nn.relu(_bdot(t, w1[...]) + b1[...])
        t = jax.nn.relu(_bdot(t, w2[...]) + b2[...])
        h = h + t

    nrm = _normalize(h).astype(bf16)

    v = jax.nn.relu(_bdot16(nrm, cw1_ref[...]) + cb1_ref[...])
    val_ref[...] = _bdot(v, cw2_ref[...]) + cb2_ref[...]

    h1 = _bdot16(nrm, hw1_ref[...]) + hb1_ref[...]

    hid = jnp.dot(oh9, p2h_ref[...]).astype(jnp.int32)
    cols = jax.lax.broadcasted_iota(jnp.int32, (1, _HSUM), 1)
    seg = jnp.zeros((1, _HSUM), jnp.int32)
    for off in _OFFS[1:-1]:
        seg = seg + (cols >= int(off)).astype(jnp.int32)
    h1m = jnp.where(seg == hid, jax.nn.relu(h1).astype(bf16), bf16(0))

    logits = _bdot16(h1m, hw2_ref[...])
    iota8 = jax.lax.broadcasted_iota(jnp.int32, (blk, 8), 1)
    oh8 = (iota8 == hid).astype(f32)
    logits = logits + jnp.dot(oh8, hb2_ref[...])

    masked = jnp.where(amask_ref[...] > 0.5, logits, f32(-1e8))
    mx = jnp.max(masked, axis=-1, keepdims=True)
    z = masked - mx
    ez = jnp.exp(z)
    s = jnp.sum(ez, axis=-1, keepdims=True)
    logp = z - jnp.log(s)

    act = act_ref[...]
    iota_a = jax.lax.broadcasted_iota(jnp.int32, (blk, logits.shape[1]), 1)
    oh_a = (act == iota_a).astype(f32)
    logp_ref[...] = jnp.sum(logp * oh_a, axis=-1, keepdims=True)
    probs = ez / s
    ent_ref[...] = -jnp.sum(probs * logp, axis=-1, keepdims=True)


@jax.jit
def _run(x, action_mask, phase_ids, action, params):
    B, OBS = x.shape
    ACT = action_mask.shape[1]
    H = _H
    PE = params['phase_embed'].shape[1]
    BLK = 512
    nb = B // BLK

    (wx, wp,
     bw1_1, bb1_1, bw2_1,
     bw1_2, bb1_2, bw2_2,
     bw1_3, bb1_3, bw2_3,
     cw1, cb1, cw2,
     hw1, hb1, hw2, hb2) = _prep(params)

    e = params['embed']
    ph2 = phase_ids.astype(jnp.int32).reshape(B, 1)
    act2 = action.astype(jnp.int32).reshape(B, 1)

    row_spec = lambda w: pl.BlockSpec((BLK, w), lambda i: (i, 0))
    full = lambda *shape: pl.BlockSpec(shape, lambda i: (0,) * len(shape))

    out_shapes = [
        jax.ShapeDtypeStruct((B, 1), jnp.float32),
        jax.ShapeDtypeStruct((B, 1), jnp.float32),
        jax.ShapeDtypeStruct((B, 1), jnp.float32),
    ]
    blkspecs = []
    for _ in range(3):
        blkspecs += [full(H, H), full(1, H), full(H, H), full(H)]
    logp, ent, val = pl.pallas_call(
        _fused_body,
        grid=(nb,),
        in_specs=[
            row_spec(OBS), row_spec(1), row_spec(1), row_spec(ACT),
            full(9, PE), full(OBS, H), full(16, H), full(H), full(H), full(H),
            *blkspecs,
            full(H, H), full(1, H), full(H, 1), full(1),
            full(H, _HSUM), full(1, _HSUM), full(_HSUM, ACT), full(8, ACT),
            full(9, 1),
        ],
        out_specs=[row_spec(1), row_spec(1), row_spec(1)],
        out_shape=out_shapes,
    )(
        x, ph2, act2, action_mask,
        params['phase_embed'], wx, wp, e['b'], e['g'], e['be'],
        bw1_1, bb1_1, bw2_1, params['blocks'][0]['b2'],
        bw1_2, bb1_2, bw2_2, params['blocks'][1]['b2'],
        bw1_3, bb1_3, bw2_3, params['blocks'][2]['b2'],
        cw1, cb1, cw2, params['critic']['b2'],
        hw1, hb1, hw2, hb2,
        jnp.asarray(_PHASE_TO_HEADIDX.astype(np.float32)[:, None]),
    )
    return action, logp[:, 0], ent[:, 0], val


def kernel(x, action_mask, phase_ids, action, params):
    return _run(x, action_mask, phase_ids, action, params)


# speedup vs baseline: 3.1697x; 1.0229x over previous
import functools

import jax
import jax.numpy as jnp
import numpy as np
from jax.experimental import pallas as pl
from jax.experimental.pallas import tpu as pltpu

_HEAD_ORDER = ['role_select', 'settler', 'builder', 'mayor', 'craftsman', 'trader', 'captain']
_HEAD_HIDDEN = [512, 256, 512, 512, 128, 256, 512]
_PHASE_TO_HEADIDX = np.array([1, 3, 2, 4, 5, 6, 6, 0, 0], dtype=np.int32)
_OFFS = np.concatenate([[0], np.cumsum(_HEAD_HIDDEN)])
_HSUM = int(_OFFS[-1])
_H = 512
_ACT = 200
_OBS = 210


def _bdot16(a, b):
    return jax.lax.dot(a, b, preferred_element_type=jnp.float32)


def _bdotb(a, b):
    return jax.lax.dot(a, b, preferred_element_type=jnp.bfloat16)


def _bdot(a, b):
    return jax.lax.dot(a.astype(jnp.bfloat16), b, preferred_element_type=jnp.float32)


def _normalize(x, eps=1e-5):
    m = jnp.mean(x, axis=-1, keepdims=True)
    v = jnp.mean(x * x, axis=-1, keepdims=True) - m * m
    return (x - m) * jax.lax.rsqrt(v + eps)


def _fused_body(*refs):
    (x_ref, ph_ref, act_ref, amask_ref,
     pe_tab_ref, wx_ref, wp_ref, be_ref, ge_ref, bee_ref,
     b1g, b1b, b1w1, b1b1, b1w2, b1b2,
     b2g, b2b, b2w1, b2b1, b2w2, b2b2,
     b3g, b3b, b3w1, b3b1, b3w2, b3b2,
     cg, cb, cw1, cb1, cw2, cb2,
     h0g, h0b, h0w1, h0b1, h0w2, h0b2,
     h1g, h1b, h1w1, h1b1, h1w2, h1b2,
     h2g, h2b, h2w1, h2b1, h2w2, h2b2,
     h3g, h3b, h3w1, h3b1, h3w2, h3b2,
     h4g, h4b, h4w1, h4b1, h4w2, h4b2,
     h5g, h5b, h5w1, h5b1, h5w2, h5b2,
     h6g, h6b, h6w1, h6b1, h6w2, h6b2,
     p2h_ref,
     logp_ref, ent_ref, val_ref,
     wxs, wps,
     bw1s_1, bb1s_1, bw2s_1,
     bw1s_2, bb1s_2, bw2s_2,
     bw1s_3, bb1s_3, bw2s_3,
     cw1s, cb1s, cw2s,
     hw1s, hb1s, hw2s, hb2s) = refs

    f32 = jnp.float32
    bf16 = jnp.bfloat16
    blk = x_ref.shape[0]

    @pl.when(pl.program_id(0) == 0)
    def _prep():
        wxs[...] = wx_ref[...].astype(bf16)
        wps[...] = wp_ref[...].astype(bf16)
        for (g_r, b_r, w1_r, b1_r, w2_r, w1_o, b1_o, w2_o) in (
            (b1g, b1b, b1w1, b1b1, b1w2, bw1s_1, bb1s_1, bw2s_1),
            (b2g, b2b, b2w1, b2b1, b2w2, bw1s_2, bb1s_2, bw2s_2),
            (b3g, b3b, b3w1, b3b1, b3w2, bw1s_3, bb1s_3, bw2s_3),
        ):
            w1 = w1_r[...]
            w1_o[...] = (g_r[...][:, None] * w1).astype(bf16)
            b1_o[...] = b1_r[...][None, :] + jnp.dot(b_r[...][None, :], w1)
            w2_o[...] = w2_r[...].astype(bf16)
        w1 = cw1[...]
        cw1s[...] = (cg[...][:, None] * w1).astype(bf16)
        cb1s[...] = cb1[...][None, :] + jnp.dot(cb[...][None, :], w1)
        cw2s[...] = cw2[...].astype(bf16)
        heads = (
            (h0g, h0b, h0w1, h0b1, h0w2, h0b2),
            (h1g, h1b, h1w1, h1b1, h1w2, h1b2),
            (h2g, h2b, h2w1, h2b1, h2w2, h2b2),
            (h3g, h3b, h3w1, h3b1, h3w2, h3b2),
            (h4g, h4b, h4w1, h4b1, h4w2, h4b2),
            (h5g, h5b, h5w1, h5b1, h5w2, h5b2),
            (h6g, h6b, h6w1, h6b1, h6w2, h6b2),
        )
        for k, (g_r, b_r, w1_r, b1_r, w2_r, b2_r) in enumerate(heads):
            off, hh = int(_OFFS[k]), _HEAD_HIDDEN[k]
            w1 = w1_r[...]
            hw1s[:, off:off + hh] = (g_r[...][:, None] * w1).astype(bf16)
            hb1s[:, off:off + hh] = b1_r[...][None, :] + jnp.dot(b_r[...][None, :], w1)
            hw2s[off:off + hh, :] = w2_r[...].astype(bf16)
            hb2s[k:k + 1, :] = b2_r[...][None, :]
        hb2s[7:8, :] = jnp.zeros((1, _ACT), f32)

    ph = ph_ref[...]
    iota9 = jax.lax.broadcasted_iota(jnp.int32, (blk, 9), 1)
    oh9 = (ph == iota9).astype(f32)
    pe = jnp.dot(oh9, pe_tab_ref[...])

    u = _bdot(x_ref[...], wxs[...]) + _bdot(pe, wps[...]) + be_ref[...]
    h = jax.nn.relu(_normalize(u) * ge_ref[...] + bee_ref[...])

    for (w1, b1, w2, b2) in (
        (bw1s_1, bb1s_1, bw2s_1, b1b2),
        (bw1s_2, bb1s_2, bw2s_2, b2b2),
        (bw1s_3, bb1s_3, bw2s_3, b3b2),
    ):
        t = _normalize(h).astype(bf16)
        t = jax.nn.relu(_bdot16(t, w1[...]) + b1[...])
        t = jax.nn.relu(_bdot(t, w2[...]) + b2[...])
        h = h + t

    nrm = _normalize(h).astype(bf16)

    v = jax.nn.relu(_bdot16(nrm, cw1s[...]) + cb1s[...])
    val_ref[...] = _bdot(v, cw2s[...]) + cb2[...]

    h1 = _bdot16(nrm, hw1s[...]) + hb1s[...]

    hid = jnp.dot(oh9, p2h_ref[...]).astype(jnp.int32)
    cols = jax.lax.broadcasted_iota(jnp.int32, (1, _HSUM), 1)
    seg = jnp.zeros((1, _HSUM), jnp.int32)
    for off in _OFFS[1:-1]:
        seg = seg + (cols >= int(off)).astype(jnp.int32)
    h1m = jnp.where(seg == hid, jax.nn.relu(h1).astype(bf16), jnp.bfloat16(0))

    logits = _bdot16(h1m, hw2s[...])
    iota8 = jax.lax.broadcasted_iota(jnp.int32, (blk, 8), 1)
    oh8 = (iota8 == hid).astype(f32)
    logits = logits + jnp.dot(oh8, hb2s[...])

    masked = jnp.where(amask_ref[...] > 0.5, logits, f32(-1e8))
    mx = jnp.max(masked, axis=-1, keepdims=True)
    z = masked - mx
    ez = jnp.exp(z)
    s = jnp.sum(ez, axis=-1, keepdims=True)
    logp = z - jnp.log(s)

    act = act_ref[...]
    iota_a = jax.lax.broadcasted_iota(jnp.int32, (blk, logits.shape[1]), 1)
    oh_a = (act == iota_a).astype(f32)
    logp_ref[...] = jnp.sum(logp * oh_a, axis=-1, keepdims=True)
    probs = ez / s
    ent_ref[...] = -jnp.sum(probs * logp, axis=-1, keepdims=True)


@jax.jit
def _run(x, action_mask, phase_ids, action, params):
    B, OBS = x.shape
    ACT = action_mask.shape[1]
    H = _H
    PE = params['phase_embed'].shape[1]
    BLK = 512
    nb = B // BLK
    bf16 = jnp.bfloat16
    f32 = jnp.float32

    e = params['embed']
    ph2 = phase_ids.astype(jnp.int32).reshape(B, 1)
    act2 = action.astype(jnp.int32).reshape(B, 1)

    row_spec = lambda w: pl.BlockSpec((BLK, w), lambda i: (i, 0))
    full = lambda *shape: pl.BlockSpec(shape, lambda i: (0,) * len(shape))

    ins = [x, ph2, act2, action_mask,
           params['phase_embed'], e['W'][:OBS], e['W'][OBS:], e['b'], e['g'], e['be']]
    in_specs = [row_spec(OBS), row_spec(1), row_spec(1), row_spec(ACT),
                full(9, PE), full(OBS, H), full(16, H), full(H), full(H), full(H)]
    for b in params['blocks']:
        ins += [b['g'], b['be'], b['W1'], b['b1'], b['W2'], b['b2']]
        in_specs += [full(H), full(H), full(H, H), full(H), full(H, H), full(H)]
    c = params['critic']
    ins += [c['g'], c['be'], c['W1'], c['b1'], c['W2'], c['b2']]
    in_specs += [full(H), full(H), full(H, H), full(H), full(H, 1), full(1)]
    for n, hh in zip(_HEAD_ORDER, _HEAD_HIDDEN):
        hp = params['heads'][n]
        ins += [hp['g'], hp['be'], hp['W1'], hp['b1'], hp['W2'], hp['b2']]
        in_specs += [full(H), full(H), full(H, hh), full(hh), full(hh, ACT), full(ACT)]
    ins += [jnp.asarray(_PHASE_TO_HEADIDX.astype(np.float32)[:, None])]
    in_specs += [full(9, 1)]

    scratch = [
        pltpu.VMEM((OBS, H), bf16), pltpu.VMEM((16, H), bf16),
    ]
    for _ in range(3):
        scratch += [pltpu.VMEM((H, H), bf16), pltpu.VMEM((1, H), f32),
                    pltpu.VMEM((H, H), bf16)]
    scratch += [pltpu.VMEM((H, H), bf16), pltpu.VMEM((1, H), f32),
                pltpu.VMEM((H, 1), bf16)]
    scratch += [pltpu.VMEM((H, _HSUM), bf16), pltpu.VMEM((1, _HSUM), f32),
                pltpu.VMEM((_HSUM, ACT), bf16), pltpu.VMEM((8, ACT), f32)]

    out_shapes = [
        jax.ShapeDtypeStruct((B, 1), f32),
        jax.ShapeDtypeStruct((B, 1), f32),
        jax.ShapeDtypeStruct((B, 1), f32),
    ]
    logp, ent, val = pl.pallas_call(
        _fused_body,
        grid=(nb,),
        in_specs=in_specs,
        out_specs=[row_spec(1), row_spec(1), row_spec(1)],
        out_shape=out_shapes,
        scratch_shapes=scratch,
    )(*ins)
    return action, logp[:, 0], ent[:, 0], val


def kernel(x, action_mask, phase_ids, action, params):
    return _run(x, action_mask, phase_ids, action, params)


# speedup vs baseline: 3.4691x; 1.0944x over previous
import functools

import jax
import jax.numpy as jnp
import numpy as np
from jax.experimental import pallas as pl
from jax.experimental.pallas import tpu as pltpu

_HEAD_ORDER = ['role_select', 'settler', 'builder', 'mayor', 'craftsman', 'trader', 'captain']
_HEAD_HIDDEN = [512, 256, 512, 512, 128, 256, 512]
_PHASE_TO_HEADIDX = np.array([1, 3, 2, 4, 5, 6, 6, 0, 0], dtype=np.int32)
_OFFS = np.concatenate([[0], np.cumsum(_HEAD_HIDDEN)])
_HSUM = int(_OFFS[-1])
_H = 512
_ACT = 200
_OBS = 210


def _bdot16(a, b):
    return jax.lax.dot(a, b, preferred_element_type=jnp.float32)


def _bdotb(a, b):
    return jax.lax.dot(a, b, preferred_element_type=jnp.bfloat16)


def _bdot(a, b):
    return jax.lax.dot(a.astype(jnp.bfloat16), b, preferred_element_type=jnp.float32)


def _normalize(x, eps=1e-5):
    m = jnp.mean(x, axis=-1, keepdims=True)
    v = jnp.mean(x * x, axis=-1, keepdims=True) - m * m
    return (x - m) * jax.lax.rsqrt(v + eps)


def _fused_body(*refs):
    (x_ref, ph_ref, act_ref, amask_ref,
     pe_tab_ref, ew_ref, be_ref, ge_ref, bee_ref,
     b1g, b1b, b1w1, b1b1, b1w2, b1b2,
     b2g, b2b, b2w1, b2b1, b2w2, b2b2,
     b3g, b3b, b3w1, b3b1, b3w2, b3b2,
     cg, cb, cw1, cb1, cw2, cb2,
     h0g, h0b, h0w1, h0b1, h0w2, h0b2,
     h1g, h1b, h1w1, h1b1, h1w2, h1b2,
     h2g, h2b, h2w1, h2b1, h2w2, h2b2,
     h3g, h3b, h3w1, h3b1, h3w2, h3b2,
     h4g, h4b, h4w1, h4b1, h4w2, h4b2,
     h5g, h5b, h5w1, h5b1, h5w2, h5b2,
     h6g, h6b, h6w1, h6b1, h6w2, h6b2,
     p2h_ref,
     logp_ref, ent_ref, val_ref,
     wxs, wps,
     bw1s_1, bb1s_1, bw2s_1,
     bw1s_2, bb1s_2, bw2s_2,
     bw1s_3, bb1s_3, bw2s_3,
     cw1s, cb1s, cw2s,
     hw1s, hb1s, hw2s, hb2s) = refs

    f32 = jnp.float32
    bf16 = jnp.bfloat16
    blk = x_ref.shape[0]

    @pl.when(pl.program_id(0) == 0)
    def _prep():
        ew = ew_ref[...]
        wxs[...] = ew[:_OBS].astype(bf16)
        wps[...] = ew[_OBS:].astype(bf16)
        for (g_r, b_r, w1_r, b1_r, w2_r, w1_o, b1_o, w2_o) in (
            (b1g, b1b, b1w1, b1b1, b1w2, bw1s_1, bb1s_1, bw2s_1),
            (b2g, b2b, b2w1, b2b1, b2w2, bw1s_2, bb1s_2, bw2s_2),
            (b3g, b3b, b3w1, b3b1, b3w2, bw1s_3, bb1s_3, bw2s_3),
        ):
            w1 = w1_r[...]
            w1_o[...] = (g_r[...][:, None] * w1).astype(bf16)
            b1_o[...] = b1_r[...][None, :] + jnp.dot(b_r[...][None, :], w1)
            w2_o[...] = w2_r[...].astype(bf16)
        w1 = cw1[...]
        cw1s[...] = (cg[...][:, None] * w1).astype(bf16)
        cb1s[...] = cb1[...][None, :] + jnp.dot(cb[...][None, :], w1)
        cw2s[...] = cw2[...].astype(bf16)
        heads = (
            (h0g, h0b, h0w1, h0b1, h0w2, h0b2),
            (h1g, h1b, h1w1, h1b1, h1w2, h1b2),
            (h2g, h2b, h2w1, h2b1, h2w2, h2b2),
            (h3g, h3b, h3w1, h3b1, h3w2, h3b2),
            (h4g, h4b, h4w1, h4b1, h4w2, h4b2),
            (h5g, h5b, h5w1, h5b1, h5w2, h5b2),
            (h6g, h6b, h6w1, h6b1, h6w2, h6b2),
        )
        for k, (g_r, b_r, w1_r, b1_r, w2_r, b2_r) in enumerate(heads):
            off, hh = int(_OFFS[k]), _HEAD_HIDDEN[k]
            w1 = w1_r[...]
            hw1s[:, off:off + hh] = (g_r[...][:, None] * w1).astype(bf16)
            hb1s[:, off:off + hh] = b1_r[...][None, :] + jnp.dot(b_r[...][None, :], w1)
            hw2s[off:off + hh, :] = w2_r[...].astype(bf16)
            hb2s[k:k + 1, :] = b2_r[...][None, :]
        hb2s[7:8, :] = jnp.zeros((1, _ACT), f32)

    ph = ph_ref[...]
    iota9 = jax.lax.broadcasted_iota(jnp.int32, (blk, 9), 1)
    oh9 = (ph == iota9).astype(f32)
    pe = jnp.dot(oh9, pe_tab_ref[...])

    u = _bdot(x_ref[...], wxs[...]) + _bdot(pe, wps[...]) + be_ref[...]
    h = jax.nn.relu(_normalize(u) * ge_ref[...] + bee_ref[...])

    for (w1, b1, w2, b2) in (
        (bw1s_1, bb1s_1, bw2s_1, b1b2),
        (bw1s_2, bb1s_2, bw2s_2, b2b2),
        (bw1s_3, bb1s_3, bw2s_3, b3b2),
    ):
        t = _normalize(h).astype(bf16)
        t = jax.nn.relu(_bdot16(t, w1[...]) + b1[...])
        t = jax.nn.relu(_bdot(t, w2[...]) + b2[...])
        h = h + t

    nrm = _normalize(h).astype(bf16)

    v = jax.nn.relu(_bdot16(nrm, cw1s[...]) + cb1s[...])
    val_ref[...] = _bdot(v, cw2s[...]) + cb2[...]

    h1 = _bdot16(nrm, hw1s[...]) + hb1s[...]

    hid = jnp.dot(oh9, p2h_ref[...]).astype(jnp.int32)
    cols = jax.lax.broadcasted_iota(jnp.int32, (1, _HSUM), 1)
    seg = jnp.zeros((1, _HSUM), jnp.int32)
    for off in _OFFS[1:-1]:
        seg = seg + (cols >= int(off)).astype(jnp.int32)
    h1m = jnp.where(seg == hid, jax.nn.relu(h1).astype(bf16), jnp.bfloat16(0))

    logits = _bdot16(h1m, hw2s[...])
    iota8 = jax.lax.broadcasted_iota(jnp.int32, (blk, 8), 1)
    oh8 = (iota8 == hid).astype(f32)
    logits = logits + jnp.dot(oh8, hb2s[...])

    masked = jnp.where(amask_ref[...] > 0.5, logits, f32(-1e8))
    mx = jnp.max(masked, axis=-1, keepdims=True)
    z = masked - mx
    ez = jnp.exp(z)
    s = jnp.sum(ez, axis=-1, keepdims=True)
    logp = z - jnp.log(s)

    act = act_ref[...]
    iota_a = jax.lax.broadcasted_iota(jnp.int32, (blk, logits.shape[1]), 1)
    oh_a = (act == iota_a).astype(f32)
    logp_ref[...] = jnp.sum(logp * oh_a, axis=-1, keepdims=True)
    probs = ez / s
    ent_ref[...] = -jnp.sum(probs * logp, axis=-1, keepdims=True)


@jax.jit
def _run(x, action_mask, phase_ids, action, params):
    B, OBS = x.shape
    ACT = action_mask.shape[1]
    H = _H
    PE = params['phase_embed'].shape[1]
    BLK = 1024
    nb = B // BLK
    bf16 = jnp.bfloat16
    f32 = jnp.float32

    e = params['embed']
    ph2 = phase_ids.astype(jnp.int32).reshape(B, 1)
    act2 = action.astype(jnp.int32).reshape(B, 1)

    row_spec = lambda w: pl.BlockSpec((BLK, w), lambda i: (i, 0))
    full = lambda *shape: pl.BlockSpec(shape, lambda i: (0,) * len(shape))

    ins = [x, ph2, act2, action_mask,
           params['phase_embed'], e['W'], e['b'], e['g'], e['be']]
    in_specs = [row_spec(OBS), row_spec(1), row_spec(1), row_spec(ACT),
                full(9, PE), full(OBS + 16, H), full(H), full(H), full(H)]
    for b in params['blocks']:
        ins += [b['g'], b['be'], b['W1'], b['b1'], b['W2'], b['b2']]
        in_specs += [full(H), full(H), full(H, H), full(H), full(H, H), full(H)]
    c = params['critic']
    ins += [c['g'], c['be'], c['W1'], c['b1'], c['W2'], c['b2']]
    in_specs += [full(H), full(H), full(H, H), full(H), full(H, 1), full(1)]
    for n, hh in zip(_HEAD_ORDER, _HEAD_HIDDEN):
        hp = params['heads'][n]
        ins += [hp['g'], hp['be'], hp['W1'], hp['b1'], hp['W2'], hp['b2']]
        in_specs += [full(H), full(H), full(H, hh), full(hh), full(hh, ACT), full(ACT)]
    ins += [jnp.asarray(_PHASE_TO_HEADIDX.astype(np.float32)[:, None])]
    in_specs += [full(9, 1)]

    scratch = [
        pltpu.VMEM((OBS, H), bf16), pltpu.VMEM((16, H), bf16),
    ]
    for _ in range(3):
        scratch += [pltpu.VMEM((H, H), bf16), pltpu.VMEM((1, H), f32),
                    pltpu.VMEM((H, H), bf16)]
    scratch += [pltpu.VMEM((H, H), bf16), pltpu.VMEM((1, H), f32),
                pltpu.VMEM((H, 1), bf16)]
    scratch += [pltpu.VMEM((H, _HSUM), bf16), pltpu.VMEM((1, _HSUM), f32),
                pltpu.VMEM((_HSUM, ACT), bf16), pltpu.VMEM((8, ACT), f32)]

    out_shapes = [
        jax.ShapeDtypeStruct((B, 1), f32),
        jax.ShapeDtypeStruct((B, 1), f32),
        jax.ShapeDtypeStruct((B, 1), f32),
    ]
    logp, ent, val = pl.pallas_call(
        _fused_body,
        grid=(nb,),
        in_specs=in_specs,
        out_specs=[row_spec(1), row_spec(1), row_spec(1)],
        out_shape=out_shapes,
        scratch_shapes=scratch,
    )(*ins)
    return action, logp[:, 0], ent[:, 0], val


def kernel(x, action_mask, phase_ids, action, params):
    return _run(x, action_mask, phase_ids, action, params)
